# baseline (device time: 46020 ns/iter reference)
import jax
import jax.numpy as jnp
from jax import lax
from jax.experimental import pallas as pl
from jax.experimental.pallas import tpu as pltpu

B, H, D, BS = 8, 8, 128, 16
NPAGES = 512
NPOS = 512
PB = 64
NB = NPAGES // PB
RK = PB * BS * H
R = B * H
SCALE = D ** -0.5
NEG = -1e30


def kernel(Q, K, V, bt, lens):
    lens2 = lens.reshape(1, B)
    k2 = K.reshape(NPAGES * BS * H, D)
    v2 = V.reshape(NPAGES * BS * H, D)

    def body(q_ref, k_ref, v_ref, bt_ref, lens_ref, out_ref,
             qf_ref, m_ref, l_ref, o_ref, wt_ref, hm_ref, send_ref, recv_ref,
             send_sem, recv_sem):
        kstep = pl.program_id(0)
        my_y = lax.axis_index("y")

        @pl.when(kstep == 0)
        def _():
            m_ref[...] = jnp.full((1, R), NEG, jnp.float32)
            l_ref[...] = jnp.zeros((1, R), jnp.float32)
            o_ref[...] = jnp.zeros((D, R), jnp.float32)
            for h in range(H):
                qf_ref[h * B:(h + 1) * B, :] = \
                    (q_ref[:, 0, h, :] * SCALE).astype(jnp.bfloat16)
            rowh = lax.broadcasted_iota(jnp.int32, (RK, R), 0) % H
            colh = lax.broadcasted_iota(jnp.int32, (RK, R), 1) // B
            hm_ref[...] = (rowh == colh).astype(jnp.float32)
            pids = my_y * NPAGES + lax.broadcasted_iota(
                jnp.int32, (NPAGES, 1), 0)
            jpos = lax.broadcasted_iota(jnp.int32, (1, NPOS), 1)
            for b in range(B):
                bt_row = bt_ref[b:b + 1, :]
                v_row = (jpos < lens_ref[0:1, b:b + 1]).astype(jnp.float32)
                eq = (pids == bt_row).astype(jnp.float32)
                wt_ref[:, b:b + 1] = jnp.sum(eq * v_row, axis=1, keepdims=True)

        wp = wt_ref[pl.ds(kstep * PB, PB), :]
        wrep = jnp.repeat(wp, BS * H, axis=0)
        w2 = jnp.concatenate([wrep] * H, axis=1)
        wmask = hm_ref[...] * w2

        kb = k_ref[...].astype(jnp.bfloat16)
        vb = v_ref[...].astype(jnp.bfloat16)
        s = lax.dot_general(kb, qf_ref[...], (((1,), (1,)), ((), ())),
                            preferred_element_type=jnp.float32)
        m_old = m_ref[...]
        m_new = jnp.maximum(m_old, jnp.max(s, axis=0, keepdims=True))
        alpha = jnp.exp(m_old - m_new)
        p = wmask * jnp.exp(s - m_new)
        l_new = alpha * l_ref[...] + jnp.sum(p, axis=0, keepdims=True)
        o_step = lax.dot_general(vb, p.astype(jnp.bfloat16),
                                 (((0,), (0,)), ((), ())),
                                 preferred_element_type=jnp.float32)
        m_ref[...] = m_new
        l_ref[...] = l_new
        o_ref[...] = alpha * o_ref[...] + o_step

        @pl.when(kstep == NB - 1)
        def _():
            send_ref[0:D, :] = o_ref[...]
            send_ref[D:D + 1, :] = m_ref[...]
            send_ref[D + 1:D + 2, :] = l_ref[...]

            my_x = lax.axis_index("x")
            my_z = lax.axis_index("z")
            rdma = pltpu.make_async_remote_copy(
                src_ref=send_ref,
                dst_ref=recv_ref,
                send_sem=send_sem,
                recv_sem=recv_sem,
                device_id=(my_x, 1 - my_y, my_z),
                device_id_type=pl.DeviceIdType.MESH,
            )
            rdma.start()
            rdma.wait()

            o_b = recv_ref[0:D, :]
            m_b = recv_ref[D:D + 1, :]
            l_b = recv_ref[D + 1:D + 2, :]
            m_a = m_ref[...]
            l_a = l_ref[...]
            o_a = o_ref[...]
            m_s = jnp.maximum(m_a, m_b)
            ea = jnp.exp(m_a - m_s)
            eb = jnp.exp(m_b - m_s)
            denom = ea * l_a + eb * l_b
            o_fin = (ea * o_a + eb * o_b) / denom
            o_t = jnp.transpose(o_fin)
            for h in range(H):
                out_ref[:, 0, h, :] = o_t[h * B:(h + 1) * B, :]

    return pl.pallas_call(
        body,
        grid=(NB,),
        in_specs=[
            pl.BlockSpec((B, 1, H, D), lambda k: (0, 0, 0, 0)),
            pl.BlockSpec((RK, D), lambda k: (k, 0)),
            pl.BlockSpec((RK, D), lambda k: (k, 0)),
            pl.BlockSpec((B, NPOS), lambda k: (0, 0)),
            pl.BlockSpec((1, B), lambda k: (0, 0)),
        ],
        out_specs=pl.BlockSpec((B, 1, H, D), lambda k: (0, 0, 0, 0)),
        out_shape=jax.ShapeDtypeStruct((B, 1, H, D), jnp.float32),
        scratch_shapes=[
            pltpu.VMEM((R, D), jnp.bfloat16),
            pltpu.VMEM((1, R), jnp.float32),
            pltpu.VMEM((1, R), jnp.float32),
            pltpu.VMEM((D, R), jnp.float32),
            pltpu.VMEM((NPAGES, B), jnp.float32),
            pltpu.VMEM((RK, R), jnp.float32),
            pltpu.VMEM((D + 2, R), jnp.float32),
            pltpu.VMEM((D + 2, R), jnp.float32),
            pltpu.SemaphoreType.DMA,
            pltpu.SemaphoreType.DMA,
        ],
        compiler_params=pltpu.CompilerParams(
            dimension_semantics=("arbitrary",),
        ),
    )(Q, k2, v2, bt, lens2)


# device time: 36565 ns/iter; 1.2586x vs baseline; 1.2586x over previous
import jax
import jax.numpy as jnp
from jax import lax
from jax.experimental import pallas as pl
from jax.experimental.pallas import tpu as pltpu

B, H, D, BS = 8, 8, 128, 16
NPAGES = 512
NPOS = 512
PB = 64
NB = NPAGES // PB
T = PB * BS
R = B * H
HD = H * D
SCALE = D ** -0.5
NEG = -1e30


def kernel(Q, K, V, bt, lens):
    lens2 = lens.reshape(1, B)

    def body(q_ref, k_any, v_any, bt_ref, lens_ref, out_ref,
             kp_ref, vp_ref, qbd_ref, m_ref, l_ref, o_ref, wt_ref,
             send_ref, recv_ref, dma_sems, send_sem, recv_sem):
        kstep = pl.program_id(0)
        my_y = lax.axis_index("y")
        slot = kstep % 2

        def dma(buf_slot, step, h, src, dst, sem_row):
            return pltpu.make_async_copy(
                src.at[pl.ds(step * PB, PB), :, h, :],
                dst.at[buf_slot, :, :, pl.ds(h * D, D)],
                dma_sems.at[buf_slot, sem_row],
            )

        def issue(buf_slot, step):
            for h in range(H):
                dma(buf_slot, step, h, k_any, kp_ref, h).start()
                dma(buf_slot, step, h, v_any, vp_ref, H + h).start()

        @pl.when(kstep == 0)
        def _():
            issue(0, 0)
            issue(1, 1)
            m_ref[...] = jnp.full((R, 1), NEG, jnp.float32)
            l_ref[...] = jnp.zeros((R, 1), jnp.float32)
            o_ref[...] = jnp.zeros((R, D), jnp.float32)
            rep = jnp.concatenate([(q_ref[...].reshape(R, D) * SCALE)] * H,
                                  axis=1)
            rmask = (lax.broadcasted_iota(jnp.int32, (R, HD), 1) // D
                     == lax.broadcasted_iota(jnp.int32, (R, HD), 0) % H)
            qbd_ref[...] = jnp.where(rmask, rep, 0.0).astype(jnp.bfloat16)
            pids = my_y * NPAGES + lax.broadcasted_iota(
                jnp.int32, (NPAGES, 1), 0)
            jpos = lax.broadcasted_iota(jnp.int32, (1, NPOS), 1)
            for b in range(B):
                bt_row = bt_ref[b:b + 1, :]
                v_row = (jpos < lens_ref[0:1, b:b + 1]).astype(jnp.float32)
                eq = (pids == bt_row).astype(jnp.float32)
                wt_ref[:, b:b + 1] = jnp.sum(eq * v_row, axis=1, keepdims=True)

        rowp = lax.broadcasted_iota(jnp.int32, (NPAGES, T), 0)
        colp = kstep * PB + lax.broadcasted_iota(jnp.int32, (NPAGES, T), 1) // BS
        expand = (rowp == colp).astype(jnp.float32)
        w_tok = lax.dot_general(wt_ref[...], expand, (((0,), (0,)), ((), ())),
                                preferred_element_type=jnp.float32)
        rsel = (lax.broadcasted_iota(jnp.int32, (R, B), 0) // H
                == lax.broadcasted_iota(jnp.int32, (R, B), 1)).astype(jnp.float32)
        w64 = lax.dot_general(rsel, w_tok, (((1,), (0,)), ((), ())),
                              preferred_element_type=jnp.float32)

        for h in range(H):
            dma(slot, kstep, h, k_any, kp_ref, h).wait()
            dma(slot, kstep, h, v_any, vp_ref, H + h).wait()

        kb = kp_ref[slot].reshape(T, HD).astype(jnp.bfloat16)
        vb = vp_ref[slot].reshape(T, HD).astype(jnp.bfloat16)
        s = lax.dot_general(qbd_ref[...], kb, (((1,), (1,)), ((), ())),
                            preferred_element_type=jnp.float32)
        m_old = m_ref[...]
        m_new = jnp.maximum(m_old, jnp.max(s, axis=1, keepdims=True))
        alpha = jnp.exp(m_old - m_new)
        p = w64 * jnp.exp(s - m_new)
        l_new = alpha * l_ref[...] + jnp.sum(p, axis=1, keepdims=True)
        o_full = lax.dot_general(p.astype(jnp.bfloat16), vb,
                                 (((1,), (0,)), ((), ())),
                                 preferred_element_type=jnp.float32)
        cmask = (lax.broadcasted_iota(jnp.int32, (R, HD), 1) // D
                 == lax.broadcasted_iota(jnp.int32, (R, HD), 0) % H)
        o_sel = jnp.where(cmask, o_full, 0.0)
        o_step = o_sel[:, 0:D]
        for j in range(1, H):
            o_step = o_step + o_sel[:, j * D:(j + 1) * D]
        m_ref[...] = m_new
        l_ref[...] = l_new
        o_ref[...] = alpha * o_ref[...] + o_step

        @pl.when(kstep + 2 < NB)
        def _():
            issue(slot, kstep + 2)

        @pl.when(kstep == NB - 1)
        def _():
            send_ref[:, 0:D] = o_ref[...]
            send_ref[:, D:D + 1] = m_ref[...]
            send_ref[:, D + 1:D + 2] = l_ref[...]

            my_x = lax.axis_index("x")
            my_z = lax.axis_index("z")
            rdma = pltpu.make_async_remote_copy(
                src_ref=send_ref,
                dst_ref=recv_ref,
                send_sem=send_sem,
                recv_sem=recv_sem,
                device_id=(my_x, 1 - my_y, my_z),
                device_id_type=pl.DeviceIdType.MESH,
            )
            rdma.start()
            rdma.wait()

            o_b = recv_ref[:, 0:D]
            m_b = recv_ref[:, D:D + 1]
            l_b = recv_ref[:, D + 1:D + 2]
            m_a = m_ref[...]
            l_a = l_ref[...]
            o_a = o_ref[...]
            m_s = jnp.maximum(m_a, m_b)
            ea = jnp.exp(m_a - m_s)
            eb = jnp.exp(m_b - m_s)
            denom = ea * l_a + eb * l_b
            out = (ea * o_a + eb * o_b) / denom
            out_ref[...] = out.reshape(B, 1, H, D)

    return pl.pallas_call(
        body,
        grid=(NB,),
        in_specs=[
            pl.BlockSpec((B, 1, H, D), lambda k: (0, 0, 0, 0)),
            pl.BlockSpec(memory_space=pl.ANY),
            pl.BlockSpec(memory_space=pl.ANY),
            pl.BlockSpec((B, NPOS), lambda k: (0, 0)),
            pl.BlockSpec((1, B), lambda k: (0, 0)),
        ],
        out_specs=pl.BlockSpec((B, 1, H, D), lambda k: (0, 0, 0, 0)),
        out_shape=jax.ShapeDtypeStruct((B, 1, H, D), jnp.float32),
        scratch_shapes=[
            pltpu.VMEM((2, PB, BS, HD), jnp.float32),
            pltpu.VMEM((2, PB, BS, HD), jnp.float32),
            pltpu.VMEM((R, HD), jnp.bfloat16),
            pltpu.VMEM((R, 1), jnp.float32),
            pltpu.VMEM((R, 1), jnp.float32),
            pltpu.VMEM((R, D), jnp.float32),
            pltpu.VMEM((NPAGES, B), jnp.float32),
            pltpu.VMEM((R, D + 2), jnp.float32),
            pltpu.VMEM((R, D + 2), jnp.float32),
            pltpu.SemaphoreType.DMA((2, 2 * H)),
            pltpu.SemaphoreType.DMA,
            pltpu.SemaphoreType.DMA,
        ],
        compiler_params=pltpu.CompilerParams(
            dimension_semantics=("arbitrary",),
        ),
    )(Q, K, V, bt, lens2)
